# Initial kernel scaffold; baseline (speedup 1.0000x reference)
import jax
import jax.numpy as jnp
from jax.experimental import pallas as pl


def kernel(node_heads, demands, edge_index, edge_attr):
    raise NotImplementedError("write your pallas kernel here")


# speedup vs baseline: 4.6065x; 4.6065x over previous
import functools

import jax
import jax.numpy as jnp
from jax import lax
from jax.experimental import pallas as pl
from jax.experimental.pallas import tpu as pltpu
from jax.experimental.pallas import tpu_sc as plsc

NC = 2
NS = 16
LANES = 16
RESERVOIR_HEAD = 50.0


def _vec_loop(n_vec, body):
    lax.fori_loop(0, n_vec, lambda j, c: (body(j), c)[1], 0)


def _build_main(B, N, E, K):
    EQ = E // 4
    EP = E // (NC * NS)
    assert EQ % K == 0 and EP % K == 0 and K % LANES == 0 and N % LANES == 0

    mesh = plsc.VectorSubcoreMesh(
        core_axis_name="c", subcore_axis_name="s",
        num_cores=NC, num_subcores=NS)

    @functools.partial(
        pl.kernel,
        out_type=[
            jax.ShapeDtypeStruct((B * E,), jnp.float32),
            jax.ShapeDtypeStruct((4 * B * N,), jnp.float32),
            jax.ShapeDtypeStruct((E,), jnp.float32),
        ],
        mesh=mesh,
        scratch_types=[
            pltpu.VMEM((N,), jnp.float32),
            pltpu.VMEM((K * 4,), jnp.float32),
            pltpu.VMEM((K,), jnp.float32),
            pltpu.VMEM((K,), jnp.int32),
            pltpu.VMEM((K,), jnp.int32),
            pltpu.VMEM((K,), jnp.float32),
        ],
        compiler_params=pltpu.CompilerParams(needs_layout_passes=False),
        interpret=False,
    )
    def main(heads_flat, edge_index_flat, edge_attr, flows, partials, gsc,
             big_buf, attr_blk, g_blk, src_blk, dst_blk, flow_blk):
        c = lax.axis_index("c")
        s = lax.axis_index("s")
        lane = lax.iota(jnp.int32, LANES)

        p0_base = c * (E // 2) + s * EP

        def p0_block(blk):
            off = p0_base + blk * K
            pltpu.sync_copy(edge_attr.at[pl.ds(off * 4, K * 4)], attr_blk)

            def vec(j):
                rows4 = (j * LANES + lane) * 4
                g16 = plsc.load_gather(attr_blk, [rows4])
                g_blk[pl.ds(j * LANES, LANES)] = g16

            _vec_loop(K // LANES, vec)
            pltpu.sync_copy(g_blk, gsc.at[pl.ds(off, K)])

        lax.fori_loop(0, EP // K, lambda i, u: (p0_block(i), u)[1], 0)
        plsc.subcore_barrier()

        b = s % B
        q = 2 * c + s // B
        qbase = q * EQ
        pltpu.sync_copy(heads_flat.at[pl.ds(b * N, N)], big_buf)

        def p1_block(blk):
            off = qbase + blk * K
            pltpu.sync_copy(edge_index_flat.at[pl.ds(off, K)], src_blk)
            pltpu.sync_copy(edge_index_flat.at[pl.ds(E + off, K)], dst_blk)
            pltpu.sync_copy(gsc.at[pl.ds(off, K)], g_blk)

            def vec(j):
                sl = pl.ds(j * LANES, LANES)
                s16 = src_blk[sl]
                d16 = dst_blk[sl]
                g16 = g_blk[sl]
                hi = plsc.load_gather(big_buf, [s16])
                hj = plsc.load_gather(big_buf, [d16])
                flow_blk[sl] = g16 * (hi - hj)

            _vec_loop(K // LANES, vec)
            pltpu.sync_copy(flow_blk, flows.at[pl.ds(b * E + off, K)])

        lax.fori_loop(0, EQ // K, lambda i, u: (p1_block(i), u)[1], 0)

        zeros16 = jnp.zeros((LANES,), jnp.float32)

        def zvec(j):
            big_buf[pl.ds(j * LANES, LANES)] = zeros16

        _vec_loop(N // LANES, zvec)

        def p2_block(blk):
            off = qbase + blk * K
            pltpu.sync_copy(edge_index_flat.at[pl.ds(off, K)], src_blk)
            pltpu.sync_copy(edge_index_flat.at[pl.ds(E + off, K)], dst_blk)
            pltpu.sync_copy(flows.at[pl.ds(b * E + off, K)], flow_blk)

            def vec(j):
                sl = pl.ds(j * LANES, LANES)
                f16 = flow_blk[sl]
                plsc.addupdate_scatter(big_buf, [src_blk[sl]], f16)
                plsc.addupdate_scatter(big_buf, [dst_blk[sl]], -f16)

            _vec_loop(K // LANES, vec)

        lax.fori_loop(0, EQ // K, lambda i, u: (p2_block(i), u)[1], 0)
        pltpu.sync_copy(big_buf, partials.at[pl.ds((q * B + b) * N, N)])

    return main


def _build_reduce(B, N, KB):
    NQ = (B * N) // (NC * NS)
    assert N % 4 == 0 and NQ == N // 4 and NQ % KB == 0
    KBP = ((KB + LANES - 1) // LANES) * LANES

    mesh = plsc.VectorSubcoreMesh(
        core_axis_name="c", subcore_axis_name="s",
        num_cores=NC, num_subcores=NS)

    @functools.partial(
        pl.kernel,
        out_type=[
            jax.ShapeDtypeStruct((NC * NS * LANES,), jnp.float32),
            jax.ShapeDtypeStruct((LANES,), jnp.float32),
        ],
        mesh=mesh,
        scratch_types=[
            pltpu.VMEM((4 * KBP,), jnp.float32),
            pltpu.VMEM((KBP,), jnp.float32),
            pltpu.VMEM((LANES,), jnp.float32),
            pltpu.VMEM((B * LANES,), jnp.float32),
        ],
        interpret=False,
    )
    def reduce(partials, demands, heads_flat, cont_out, bound_out,
               p_blk, d_blk, acc_buf, bbuf):
        c = lax.axis_index("c")
        s = lax.axis_index("s")
        w = c * NS + s
        b = w // 4
        nbase = b * N + (w % 4) * NQ
        zeros16 = jnp.zeros((LANES,), jnp.float32)

        if KBP != KB:
            for qi in range(4):
                p_blk[pl.ds(qi * KBP + KBP - LANES, LANES)] = zeros16
            d_blk[pl.ds(KBP - LANES, LANES)] = zeros16

        def block(blk, acc):
            off = nbase + blk * KB
            for qi in range(4):
                pltpu.sync_copy(partials.at[pl.ds(qi * B * N + off, KB)],
                                p_blk.at[pl.ds(qi * KBP, KB)])
            pltpu.sync_copy(demands.at[pl.ds(off, KB)], d_blk.at[pl.ds(0, KB)])

            def vec(j, a):
                base = j * LANES
                v = ((p_blk[pl.ds(base, LANES)]
                      + p_blk[pl.ds(KBP + base, LANES)])
                     + (p_blk[pl.ds(2 * KBP + base, LANES)]
                        + p_blk[pl.ds(3 * KBP + base, LANES)])) \
                    - d_blk[pl.ds(base, LANES)]
                return a + v * v

            return lax.fori_loop(0, KBP // LANES, vec, acc)

        acc = lax.fori_loop(0, NQ // KB, block, zeros16)
        acc_buf[...] = acc
        pltpu.sync_copy(acc_buf, cont_out.at[pl.ds(w * LANES, LANES)])

        @pl.when(w == 0)
        def _():
            for bi in range(B):
                pltpu.sync_copy(heads_flat.at[pl.ds(bi * N, LANES)],
                                bbuf.at[pl.ds(bi * LANES, LANES)])
            lane = lax.iota(jnp.int32, LANES)
            m4 = lane < 4
            bacc = zeros16
            for bi in range(B):
                r = bbuf[pl.ds(bi * LANES, LANES)] - RESERVOIR_HEAD
                bacc = bacc + jnp.where(m4, r * r, 0.0)
            acc_buf[...] = bacc
            pltpu.sync_copy(acc_buf, bound_out)

    return reduce


def kernel(node_heads, demands, edge_index, edge_attr):
    B, N = node_heads.shape
    E = edge_index.shape[1]
    main = _build_main(B, N, E, K=2000)
    reduce = _build_reduce(B, N, KB=1000)
    heads_flat = node_heads.reshape(B * N)
    flows_flat, partials, _ = main(
        heads_flat, edge_index.reshape(2 * E), edge_attr.reshape(E * 4))
    cont_p, bound_p = reduce(partials, demands.reshape(B * N), heads_flat)
    continuity = jnp.sum(cont_p) / (B * N)
    boundary = jnp.sum(bound_p) / (B * 4)
    total = continuity + boundary
    return (continuity, boundary, total, flows_flat.reshape(B, E))


# speedup vs baseline: 17.6577x; 3.8332x over previous
import functools

import jax
import jax.numpy as jnp
from jax import lax
from jax.experimental import pallas as pl
from jax.experimental.pallas import tpu as pltpu
from jax.experimental.pallas import tpu_sc as plsc

NC = 2
NS = 16
LANES = 16
RESERVOIR_HEAD = 50.0


def _vec_loop(n_vec, body):
    lax.fori_loop(0, n_vec, lambda j, c: (body(j), c)[1], 0)


def _build_main(B, N, E, K):
    EQ = E // 4
    assert EQ % K == 0 and K % LANES == 0 and N % LANES == 0

    mesh = plsc.VectorSubcoreMesh(
        core_axis_name="c", subcore_axis_name="s",
        num_cores=NC, num_subcores=NS)

    @functools.partial(
        pl.kernel,
        out_type=[
            jax.ShapeDtypeStruct((B * E,), jnp.float32),
            jax.ShapeDtypeStruct((4 * B * N,), jnp.float32),
        ],
        mesh=mesh,
        scratch_types=[
            pltpu.VMEM((N,), jnp.float32),
            pltpu.VMEM((K,), jnp.float32),
            pltpu.VMEM((K,), jnp.int32),
            pltpu.VMEM((K,), jnp.int32),
            pltpu.VMEM((K,), jnp.float32),
        ],
        compiler_params=pltpu.CompilerParams(needs_layout_passes=False),
        interpret=False,
    )
    def main(heads_flat, edge_index_flat, gsc, flows, partials,
             big_buf, g_blk, src_blk, dst_blk, flow_blk):
        c = lax.axis_index("c")
        s = lax.axis_index("s")

        b = s % B
        q = 2 * c + s // B
        qbase = q * EQ
        pltpu.sync_copy(heads_flat.at[pl.ds(b * N, N)], big_buf)

        def p1_block(blk):
            off = qbase + blk * K
            pltpu.sync_copy(edge_index_flat.at[pl.ds(off, K)], src_blk)
            pltpu.sync_copy(edge_index_flat.at[pl.ds(E + off, K)], dst_blk)
            pltpu.sync_copy(gsc.at[pl.ds(off, K)], g_blk)

            def vec(j):
                sl = pl.ds(j * LANES, LANES)
                s16 = src_blk[sl]
                d16 = dst_blk[sl]
                g16 = g_blk[sl]
                hi = plsc.load_gather(big_buf, [s16])
                hj = plsc.load_gather(big_buf, [d16])
                flow_blk[sl] = g16 * (hi - hj)

            _vec_loop(K // LANES, vec)
            pltpu.sync_copy(flow_blk, flows.at[pl.ds(b * E + off, K)])

        lax.fori_loop(0, EQ // K, lambda i, u: (p1_block(i), u)[1], 0)

        zeros16 = jnp.zeros((LANES,), jnp.float32)

        def zvec(j):
            big_buf[pl.ds(j * LANES, LANES)] = zeros16

        _vec_loop(N // LANES, zvec)

        def p2_block(blk):
            off = qbase + blk * K
            pltpu.sync_copy(edge_index_flat.at[pl.ds(off, K)], src_blk)
            pltpu.sync_copy(edge_index_flat.at[pl.ds(E + off, K)], dst_blk)
            pltpu.sync_copy(flows.at[pl.ds(b * E + off, K)], flow_blk)

            def vec(j):
                sl = pl.ds(j * LANES, LANES)
                f16 = flow_blk[sl]
                plsc.addupdate_scatter(big_buf, [src_blk[sl]], f16)
                plsc.addupdate_scatter(big_buf, [dst_blk[sl]], -f16)

            _vec_loop(K // LANES, vec)

        lax.fori_loop(0, EQ // K, lambda i, u: (p2_block(i), u)[1], 0)
        pltpu.sync_copy(big_buf, partials.at[pl.ds((q * B + b) * N, N)])

    return main


def _build_reduce(B, N, KB):
    NQ = (B * N) // (NC * NS)
    assert N % 4 == 0 and NQ == N // 4 and NQ % KB == 0
    KBP = ((KB + LANES - 1) // LANES) * LANES

    mesh = plsc.VectorSubcoreMesh(
        core_axis_name="c", subcore_axis_name="s",
        num_cores=NC, num_subcores=NS)

    @functools.partial(
        pl.kernel,
        out_type=[
            jax.ShapeDtypeStruct((NC * NS * LANES,), jnp.float32),
            jax.ShapeDtypeStruct((LANES,), jnp.float32),
        ],
        mesh=mesh,
        scratch_types=[
            pltpu.VMEM((4 * KBP,), jnp.float32),
            pltpu.VMEM((KBP,), jnp.float32),
            pltpu.VMEM((LANES,), jnp.float32),
            pltpu.VMEM((B * LANES,), jnp.float32),
        ],
        interpret=False,
    )
    def reduce(partials, demands, heads_flat, cont_out, bound_out,
               p_blk, d_blk, acc_buf, bbuf):
        c = lax.axis_index("c")
        s = lax.axis_index("s")
        w = c * NS + s
        b = w // 4
        nbase = b * N + (w % 4) * NQ
        zeros16 = jnp.zeros((LANES,), jnp.float32)

        if KBP != KB:
            for qi in range(4):
                p_blk[pl.ds(qi * KBP + KBP - LANES, LANES)] = zeros16
            d_blk[pl.ds(KBP - LANES, LANES)] = zeros16

        def block(blk, acc):
            off = nbase + blk * KB
            for qi in range(4):
                pltpu.sync_copy(partials.at[pl.ds(qi * B * N + off, KB)],
                                p_blk.at[pl.ds(qi * KBP, KB)])
            pltpu.sync_copy(demands.at[pl.ds(off, KB)], d_blk.at[pl.ds(0, KB)])

            def vec(j, a):
                base = j * LANES
                v = ((p_blk[pl.ds(base, LANES)]
                      + p_blk[pl.ds(KBP + base, LANES)])
                     + (p_blk[pl.ds(2 * KBP + base, LANES)]
                        + p_blk[pl.ds(3 * KBP + base, LANES)])) \
                    - d_blk[pl.ds(base, LANES)]
                return a + v * v

            return lax.fori_loop(0, KBP // LANES, vec, acc)

        acc = lax.fori_loop(0, NQ // KB, block, zeros16)
        acc_buf[...] = acc
        pltpu.sync_copy(acc_buf, cont_out.at[pl.ds(w * LANES, LANES)])

        @pl.when(w == 0)
        def _():
            for bi in range(B):
                pltpu.sync_copy(heads_flat.at[pl.ds(bi * N, LANES)],
                                bbuf.at[pl.ds(bi * LANES, LANES)])
            lane = lax.iota(jnp.int32, LANES)
            m4 = lane < 4
            bacc = zeros16
            for bi in range(B):
                r = bbuf[pl.ds(bi * LANES, LANES)] - RESERVOIR_HEAD
                bacc = bacc + jnp.where(m4, r * r, 0.0)
            acc_buf[...] = bacc
            pltpu.sync_copy(acc_buf, bound_out)

    return reduce


def kernel(node_heads, demands, edge_index, edge_attr):
    B, N = node_heads.shape
    E = edge_index.shape[1]
    main = _build_main(B, N, E, K=4000)
    reduce = _build_reduce(B, N, KB=1000)
    heads_flat = node_heads.reshape(B * N)
    flows_flat, partials = main(
        heads_flat, edge_index.reshape(2 * E), edge_attr[:, 0])
    cont_p, bound_p = reduce(partials, demands.reshape(B * N), heads_flat)
    continuity = jnp.sum(cont_p) / (B * N)
    boundary = jnp.sum(bound_p) / (B * 4)
    total = continuity + boundary
    flows = jnp.stack([flows_flat[i * E:(i + 1) * E] for i in range(B)])
    return (continuity, boundary, total, flows)


# speedup vs baseline: 28.8938x; 1.6363x over previous
import functools

import jax
import jax.numpy as jnp
from jax import lax
from jax.experimental import pallas as pl
from jax.experimental.pallas import tpu as pltpu
from jax.experimental.pallas import tpu_sc as plsc

NC = 2
NS = 16
LANES = 16
RESERVOIR_HEAD = 50.0


def _vec_loop(n_vec, body):
    lax.fori_loop(0, n_vec, lambda j, c: (body(j), c)[1], 0)


def _build_main(B, N, E, K):
    EQ = E // 4
    assert EQ % K == 0 and K % LANES == 0 and N % LANES == 0

    mesh = plsc.VectorSubcoreMesh(
        core_axis_name="c", subcore_axis_name="s",
        num_cores=NC, num_subcores=NS)

    NB = EQ // K
    assert NB % 2 == 0
    NO = NB // 2

    @functools.partial(
        pl.kernel,
        out_type=[
            jax.ShapeDtypeStruct((B * E,), jnp.float32),
            jax.ShapeDtypeStruct((4 * B * N,), jnp.float32),
        ],
        mesh=mesh,
        scratch_types=[
            pltpu.VMEM((N,), jnp.float32),
            pltpu.VMEM((K,), jnp.float32),
            pltpu.VMEM((K,), jnp.float32),
            pltpu.VMEM((K,), jnp.int32),
            pltpu.VMEM((K,), jnp.int32),
            pltpu.VMEM((K,), jnp.int32),
            pltpu.VMEM((K,), jnp.int32),
            pltpu.VMEM((K,), jnp.float32),
            pltpu.VMEM((K,), jnp.float32),
            pltpu.SemaphoreType.DMA,
            pltpu.SemaphoreType.DMA,
            pltpu.SemaphoreType.DMA,
            pltpu.SemaphoreType.DMA,
        ],
        compiler_params=pltpu.CompilerParams(needs_layout_passes=False),
        interpret=False,
    )
    def main(heads_flat, edge_index_flat, gsc, flows, partials,
             big_buf, g0, g1, s0, s1, d0, d1, f0, f1,
             lsem0, lsem1, ssem0, ssem1):
        c = lax.axis_index("c")
        s = lax.axis_index("s")
        b = s % B
        q = 2 * c + s // B
        qbase = q * EQ
        fbase = b * E + qbase
        bufs = ((s0, d0, g0, f0, lsem0, ssem0),
                (s1, d1, g1, f1, lsem1, ssem1))

        def start_loads1(blk, sb, db, gb, sem):
            off = qbase + blk * K
            pltpu.async_copy(edge_index_flat.at[pl.ds(off, K)], sb, sem)
            pltpu.async_copy(edge_index_flat.at[pl.ds(E + off, K)], db, sem)
            pltpu.async_copy(gsc.at[pl.ds(off, K)], gb, sem)

        def wait_loads(sb, db, gb, sem):
            pltpu.make_async_copy(edge_index_flat.at[pl.ds(0, K)], sb, sem).wait()
            pltpu.make_async_copy(edge_index_flat.at[pl.ds(0, K)], db, sem).wait()
            pltpu.make_async_copy(gsc.at[pl.ds(0, K)], gb, sem).wait()

        def wait_store(fb, sem):
            pltpu.make_async_copy(fb, flows.at[pl.ds(0, K)], sem).wait()

        def compute1(blk, sb, db, gb, fb, sem):
            def vec(j):
                sl = pl.ds(j * LANES, LANES)
                hi = plsc.load_gather(big_buf, [sb[sl]])
                hj = plsc.load_gather(big_buf, [db[sl]])
                fb[sl] = gb[sl] * (hi - hj)

            _vec_loop(K // LANES, vec)
            pltpu.async_copy(fb, flows.at[pl.ds(fbase + blk * K, K)], sem)

        start_loads1(0, s0, d0, g0, lsem0)
        pltpu.sync_copy(heads_flat.at[pl.ds(b * N, N)], big_buf)

        def outer1(o, u):
            for p, (sb, db, gb, fb, lsem, ssem) in enumerate(bufs):
                blk = 2 * o + p
                nsb, ndb, ngb, _, nlsem, _ = bufs[1 - p]

                @pl.when(blk + 1 < NB)
                def _():
                    start_loads1(blk + 1, nsb, ndb, ngb, nlsem)

                wait_loads(sb, db, gb, lsem)

                @pl.when(o > 0)
                def _():
                    wait_store(fb, ssem)

                compute1(blk, sb, db, gb, fb, ssem)
            return u

        lax.fori_loop(0, NO, outer1, 0)
        wait_store(f0, ssem0)
        wait_store(f1, ssem1)

        def start_loads2(blk, sb, db, fb, sem):
            off = blk * K
            pltpu.async_copy(edge_index_flat.at[pl.ds(qbase + off, K)], sb, sem)
            pltpu.async_copy(
                edge_index_flat.at[pl.ds(E + qbase + off, K)], db, sem)
            pltpu.async_copy(flows.at[pl.ds(fbase + off, K)], fb, sem)

        def wait_loads2(sb, db, fb, sem):
            pltpu.make_async_copy(edge_index_flat.at[pl.ds(0, K)], sb, sem).wait()
            pltpu.make_async_copy(edge_index_flat.at[pl.ds(0, K)], db, sem).wait()
            pltpu.make_async_copy(flows.at[pl.ds(0, K)], fb, sem).wait()

        start_loads2(0, s0, d0, f0, lsem0)

        zeros16 = jnp.zeros((LANES,), jnp.float32)

        def zvec(j):
            big_buf[pl.ds(j * LANES, LANES)] = zeros16

        _vec_loop(N // LANES, zvec)

        def outer2(o, u):
            for p, (sb, db, gb, fb, lsem, ssem) in enumerate(bufs):
                blk = 2 * o + p
                nsb, ndb, _, nfb, nlsem, _ = bufs[1 - p]

                @pl.when(blk + 1 < NB)
                def _():
                    start_loads2(blk + 1, nsb, ndb, nfb, nlsem)

                wait_loads2(sb, db, fb, lsem)

                def vec(j):
                    sl = pl.ds(j * LANES, LANES)
                    f16 = fb[sl]
                    plsc.addupdate_scatter(big_buf, [sb[sl]], f16)
                    plsc.addupdate_scatter(big_buf, [db[sl]], -f16)

                _vec_loop(K // LANES, vec)
            return u

        lax.fori_loop(0, NO, outer2, 0)
        pltpu.sync_copy(big_buf, partials.at[pl.ds((q * B + b) * N, N)])

    return main


def _build_reduce(B, N, KB):
    NQ = (B * N) // (NC * NS)
    assert N % 4 == 0 and NQ == N // 4 and NQ % KB == 0
    KBP = ((KB + LANES - 1) // LANES) * LANES

    mesh = plsc.VectorSubcoreMesh(
        core_axis_name="c", subcore_axis_name="s",
        num_cores=NC, num_subcores=NS)

    @functools.partial(
        pl.kernel,
        out_type=[
            jax.ShapeDtypeStruct((NC * NS * LANES,), jnp.float32),
            jax.ShapeDtypeStruct((LANES,), jnp.float32),
        ],
        mesh=mesh,
        scratch_types=[
            pltpu.VMEM((4 * KBP,), jnp.float32),
            pltpu.VMEM((KBP,), jnp.float32),
            pltpu.VMEM((LANES,), jnp.float32),
            pltpu.VMEM((B * LANES,), jnp.float32),
        ],
        interpret=False,
    )
    def reduce(partials, demands, heads_flat, cont_out, bound_out,
               p_blk, d_blk, acc_buf, bbuf):
        c = lax.axis_index("c")
        s = lax.axis_index("s")
        w = c * NS + s
        b = w // 4
        nbase = b * N + (w % 4) * NQ
        zeros16 = jnp.zeros((LANES,), jnp.float32)

        if KBP != KB:
            for qi in range(4):
                p_blk[pl.ds(qi * KBP + KBP - LANES, LANES)] = zeros16
            d_blk[pl.ds(KBP - LANES, LANES)] = zeros16

        def block(blk, acc):
            off = nbase + blk * KB
            for qi in range(4):
                pltpu.sync_copy(partials.at[pl.ds(qi * B * N + off, KB)],
                                p_blk.at[pl.ds(qi * KBP, KB)])
            pltpu.sync_copy(demands.at[pl.ds(off, KB)], d_blk.at[pl.ds(0, KB)])

            def vec(j, a):
                base = j * LANES
                v = ((p_blk[pl.ds(base, LANES)]
                      + p_blk[pl.ds(KBP + base, LANES)])
                     + (p_blk[pl.ds(2 * KBP + base, LANES)]
                        + p_blk[pl.ds(3 * KBP + base, LANES)])) \
                    - d_blk[pl.ds(base, LANES)]
                return a + v * v

            return lax.fori_loop(0, KBP // LANES, vec, acc)

        acc = lax.fori_loop(0, NQ // KB, block, zeros16)
        acc_buf[...] = acc
        pltpu.sync_copy(acc_buf, cont_out.at[pl.ds(w * LANES, LANES)])

        @pl.when(w == 0)
        def _():
            for bi in range(B):
                pltpu.sync_copy(heads_flat.at[pl.ds(bi * N, LANES)],
                                bbuf.at[pl.ds(bi * LANES, LANES)])
            lane = lax.iota(jnp.int32, LANES)
            m4 = lane < 4
            bacc = zeros16
            for bi in range(B):
                r = bbuf[pl.ds(bi * LANES, LANES)] - RESERVOIR_HEAD
                bacc = bacc + jnp.where(m4, r * r, 0.0)
            acc_buf[...] = bacc
            pltpu.sync_copy(acc_buf, bound_out)

    return reduce


def kernel(node_heads, demands, edge_index, edge_attr):
    B, N = node_heads.shape
    E = edge_index.shape[1]
    main = _build_main(B, N, E, K=2000)
    reduce = _build_reduce(B, N, KB=1000)
    heads_flat = node_heads.reshape(B * N)
    flows_flat, partials = main(
        heads_flat, edge_index.reshape(2 * E), edge_attr[:, 0])
    cont_p, bound_p = reduce(partials, demands.reshape(B * N), heads_flat)
    continuity = jnp.sum(cont_p) / (B * N)
    boundary = jnp.sum(bound_p) / (B * 4)
    total = continuity + boundary
    flows = jnp.stack([flows_flat[i * E:(i + 1) * E] for i in range(B)])
    return (continuity, boundary, total, flows)


# speedup vs baseline: 41.4903x; 1.4360x over previous
import functools

import jax
import jax.numpy as jnp
from jax import lax
from jax.experimental import pallas as pl
from jax.experimental.pallas import tpu as pltpu
from jax.experimental.pallas import tpu_sc as plsc

NC = 2
NS = 16
LANES = 16
RESERVOIR_HEAD = 50.0


def _vec_loop(n_vec, body):
    lax.fori_loop(0, n_vec, lambda j, c: (body(j), c)[1], 0)


def _build_main(B, N, E, K):
    EQ = E // 4
    assert EQ % K == 0 and K % LANES == 0 and N % LANES == 0

    mesh = plsc.VectorSubcoreMesh(
        core_axis_name="c", subcore_axis_name="s",
        num_cores=NC, num_subcores=NS)

    NB = EQ // K
    assert NB % 2 == 0
    NO = NB // 2

    @functools.partial(
        pl.kernel,
        out_type=[
            jax.ShapeDtypeStruct((B * E,), jnp.float32),
            jax.ShapeDtypeStruct((4 * B * N,), jnp.float32),
        ],
        mesh=mesh,
        scratch_types=[
            pltpu.VMEM((N,), jnp.float32),
            pltpu.VMEM((K,), jnp.float32),
            pltpu.VMEM((K,), jnp.float32),
            pltpu.VMEM((K,), jnp.int32),
            pltpu.VMEM((K,), jnp.int32),
            pltpu.VMEM((K,), jnp.int32),
            pltpu.VMEM((K,), jnp.int32),
            pltpu.VMEM((K,), jnp.float32),
            pltpu.VMEM((K,), jnp.float32),
            pltpu.SemaphoreType.DMA,
            pltpu.SemaphoreType.DMA,
            pltpu.SemaphoreType.DMA,
            pltpu.SemaphoreType.DMA,
        ],
        compiler_params=pltpu.CompilerParams(needs_layout_passes=False),
        interpret=False,
    )
    def main(heads_flat, edge_index_flat, gsc, flows, partials,
             big_buf, g0, g1, s0, s1, d0, d1, f0, f1,
             lsem0, lsem1, ssem0, ssem1):
        c = lax.axis_index("c")
        s = lax.axis_index("s")
        b = s % B
        q = 2 * c + s // B
        qbase = q * EQ
        fbase = b * E + qbase
        bufs = ((s0, d0, g0, f0, lsem0, ssem0),
                (s1, d1, g1, f1, lsem1, ssem1))

        def start_loads1(blk, sb, db, gb, sem):
            off = qbase + blk * K
            pltpu.async_copy(edge_index_flat.at[pl.ds(off, K)], sb, sem)
            pltpu.async_copy(edge_index_flat.at[pl.ds(E + off, K)], db, sem)
            pltpu.async_copy(gsc.at[pl.ds(off, K)], gb, sem)

        def wait_loads(sb, db, gb, sem):
            pltpu.make_async_copy(edge_index_flat.at[pl.ds(0, K)], sb, sem).wait()
            pltpu.make_async_copy(edge_index_flat.at[pl.ds(0, K)], db, sem).wait()
            pltpu.make_async_copy(gsc.at[pl.ds(0, K)], gb, sem).wait()

        def wait_store(fb, sem):
            pltpu.make_async_copy(fb, flows.at[pl.ds(0, K)], sem).wait()

        def compute1(blk, sb, db, gb, fb, sem):
            @plsc.parallel_loop(0, K, step=LANES, unroll=4)
            def _(i):
                sl = pl.ds(i, LANES)
                hi = plsc.load_gather(big_buf, [sb[sl]])
                hj = plsc.load_gather(big_buf, [db[sl]])
                fb[sl] = gb[sl] * (hi - hj)

            pltpu.async_copy(fb, flows.at[pl.ds(fbase + blk * K, K)], sem)

        start_loads1(0, s0, d0, g0, lsem0)
        pltpu.sync_copy(heads_flat.at[pl.ds(b * N, N)], big_buf)

        def outer1(o, u):
            for p, (sb, db, gb, fb, lsem, ssem) in enumerate(bufs):
                blk = 2 * o + p
                nsb, ndb, ngb, _, nlsem, _ = bufs[1 - p]

                @pl.when(blk + 1 < NB)
                def _():
                    start_loads1(blk + 1, nsb, ndb, ngb, nlsem)

                wait_loads(sb, db, gb, lsem)

                @pl.when(o > 0)
                def _():
                    wait_store(fb, ssem)

                compute1(blk, sb, db, gb, fb, ssem)
            return u

        lax.fori_loop(0, NO, outer1, 0)
        wait_store(f0, ssem0)
        wait_store(f1, ssem1)

        def start_loads2(blk, sb, db, fb, sem):
            off = blk * K
            pltpu.async_copy(edge_index_flat.at[pl.ds(qbase + off, K)], sb, sem)
            pltpu.async_copy(
                edge_index_flat.at[pl.ds(E + qbase + off, K)], db, sem)
            pltpu.async_copy(flows.at[pl.ds(fbase + off, K)], fb, sem)

        def wait_loads2(sb, db, fb, sem):
            pltpu.make_async_copy(edge_index_flat.at[pl.ds(0, K)], sb, sem).wait()
            pltpu.make_async_copy(edge_index_flat.at[pl.ds(0, K)], db, sem).wait()
            pltpu.make_async_copy(flows.at[pl.ds(0, K)], fb, sem).wait()

        start_loads2(0, s0, d0, f0, lsem0)

        zeros16 = jnp.zeros((LANES,), jnp.float32)

        @plsc.parallel_loop(0, N, step=LANES, unroll=8)
        def _(i):
            big_buf[pl.ds(i, LANES)] = zeros16

        def outer2(o, u):
            for p, (sb, db, gb, fb, lsem, ssem) in enumerate(bufs):
                blk = 2 * o + p
                nsb, ndb, _, nfb, nlsem, _ = bufs[1 - p]

                @pl.when(blk + 1 < NB)
                def _():
                    start_loads2(blk + 1, nsb, ndb, nfb, nlsem)

                wait_loads2(sb, db, fb, lsem)

                @plsc.parallel_loop(0, K, step=LANES, unroll=4)
                def _(i):
                    sl = pl.ds(i, LANES)
                    f16 = fb[sl]
                    plsc.addupdate_scatter(big_buf, [sb[sl]], f16)
                    plsc.addupdate_scatter(big_buf, [db[sl]], -f16)
            return u

        lax.fori_loop(0, NO, outer2, 0)
        pltpu.sync_copy(big_buf, partials.at[pl.ds((q * B + b) * N, N)])

    return main


def _build_reduce(B, N, KB):
    NQ = (B * N) // (NC * NS)
    assert N % 4 == 0 and NQ == N // 4 and NQ % KB == 0
    KBP = ((KB + LANES - 1) // LANES) * LANES

    mesh = plsc.VectorSubcoreMesh(
        core_axis_name="c", subcore_axis_name="s",
        num_cores=NC, num_subcores=NS)

    @functools.partial(
        pl.kernel,
        out_type=[
            jax.ShapeDtypeStruct((NC * NS * LANES,), jnp.float32),
            jax.ShapeDtypeStruct((LANES,), jnp.float32),
        ],
        mesh=mesh,
        scratch_types=[
            pltpu.VMEM((4 * KBP,), jnp.float32),
            pltpu.VMEM((KBP,), jnp.float32),
            pltpu.VMEM((LANES,), jnp.float32),
            pltpu.VMEM((B * LANES,), jnp.float32),
        ],
        interpret=False,
    )
    def reduce(partials, demands, heads_flat, cont_out, bound_out,
               p_blk, d_blk, acc_buf, bbuf):
        c = lax.axis_index("c")
        s = lax.axis_index("s")
        w = c * NS + s
        b = w // 4
        nbase = b * N + (w % 4) * NQ
        zeros16 = jnp.zeros((LANES,), jnp.float32)

        if KBP != KB:
            for qi in range(4):
                p_blk[pl.ds(qi * KBP + KBP - LANES, LANES)] = zeros16
            d_blk[pl.ds(KBP - LANES, LANES)] = zeros16

        def block(blk, acc):
            off = nbase + blk * KB
            for qi in range(4):
                pltpu.sync_copy(partials.at[pl.ds(qi * B * N + off, KB)],
                                p_blk.at[pl.ds(qi * KBP, KB)])
            pltpu.sync_copy(demands.at[pl.ds(off, KB)], d_blk.at[pl.ds(0, KB)])

            def vec(j, a):
                base = j * LANES
                v = ((p_blk[pl.ds(base, LANES)]
                      + p_blk[pl.ds(KBP + base, LANES)])
                     + (p_blk[pl.ds(2 * KBP + base, LANES)]
                        + p_blk[pl.ds(3 * KBP + base, LANES)])) \
                    - d_blk[pl.ds(base, LANES)]
                return a + v * v

            return lax.fori_loop(0, KBP // LANES, vec, acc)

        acc = lax.fori_loop(0, NQ // KB, block, zeros16)
        acc_buf[...] = acc
        pltpu.sync_copy(acc_buf, cont_out.at[pl.ds(w * LANES, LANES)])

        @pl.when(w == 0)
        def _():
            for bi in range(B):
                pltpu.sync_copy(heads_flat.at[pl.ds(bi * N, LANES)],
                                bbuf.at[pl.ds(bi * LANES, LANES)])
            lane = lax.iota(jnp.int32, LANES)
            m4 = lane < 4
            bacc = zeros16
            for bi in range(B):
                r = bbuf[pl.ds(bi * LANES, LANES)] - RESERVOIR_HEAD
                bacc = bacc + jnp.where(m4, r * r, 0.0)
            acc_buf[...] = bacc
            pltpu.sync_copy(acc_buf, bound_out)

    return reduce


def kernel(node_heads, demands, edge_index, edge_attr):
    B, N = node_heads.shape
    E = edge_index.shape[1]
    main = _build_main(B, N, E, K=2000)
    reduce = _build_reduce(B, N, KB=1000)
    heads_flat = node_heads.reshape(B * N)
    flows_flat, partials = main(
        heads_flat, edge_index.reshape(2 * E), edge_attr[:, 0])
    cont_p, bound_p = reduce(partials, demands.reshape(B * N), heads_flat)
    continuity = jnp.sum(cont_p) / (B * N)
    boundary = jnp.sum(bound_p) / (B * 4)
    total = continuity + boundary
    flows = jnp.stack([flows_flat[i * E:(i + 1) * E] for i in range(B)])
    return (continuity, boundary, total, flows)


# speedup vs baseline: 64.4997x; 1.5546x over previous
import functools

import jax
import jax.numpy as jnp
from jax import lax
from jax.experimental import pallas as pl
from jax.experimental.pallas import tpu as pltpu
from jax.experimental.pallas import tpu_sc as plsc

NC = 2
NS = 16
LANES = 16
RESERVOIR_HEAD = 50.0


def _vec_loop(n_vec, body):
    lax.fori_loop(0, n_vec, lambda j, c: (body(j), c)[1], 0)


def _build_main(B, N, E, K):
    EQ = E // 4
    assert EQ % K == 0 and K % LANES == 0 and N % LANES == 0

    mesh = plsc.VectorSubcoreMesh(
        core_axis_name="c", subcore_axis_name="s",
        num_cores=NC, num_subcores=NS)

    NB = EQ // K
    assert NB % 2 == 0 and K % 128 == 0
    NO = NB // 2
    NR = K // 128

    @functools.partial(
        pl.kernel,
        out_type=[
            jax.ShapeDtypeStruct((B * E,), jnp.float32),
            jax.ShapeDtypeStruct((4 * B * N,), jnp.float32),
        ],
        mesh=mesh,
        scratch_types=[
            pltpu.VMEM((N,), jnp.float32),
            pltpu.VMEM((K,), jnp.float32),
            pltpu.VMEM((K,), jnp.float32),
            pltpu.VMEM((K,), jnp.int32),
            pltpu.VMEM((K,), jnp.int32),
            pltpu.VMEM((K,), jnp.int32),
            pltpu.VMEM((K,), jnp.int32),
            pltpu.VMEM((K,), jnp.float32),
            pltpu.VMEM((K,), jnp.float32),
            pltpu.SemaphoreType.DMA,
            pltpu.SemaphoreType.DMA,
            pltpu.SemaphoreType.DMA,
            pltpu.SemaphoreType.DMA,
        ],
        compiler_params=pltpu.CompilerParams(needs_layout_passes=False),
        interpret=False,
    )
    def main(heads_flat, edge_index_flat, gsc, flows, partials,
             big_buf, g0, g1, s0, s1, d0, d1, f0, f1,
             lsem0, lsem1, ssem0, ssem1):
        c = lax.axis_index("c")
        s = lax.axis_index("s")
        b = s % B
        q = 2 * c + s // B
        qbase = q * EQ
        tbase = (qbase // 128) * (B * 128) + b * 128
        bufs = ((s0, d0, g0, f0, lsem0, ssem0),
                (s1, d1, g1, f1, lsem1, ssem1))

        def start_loads1(blk, sb, db, gb, sem):
            off = qbase + blk * K
            pltpu.async_copy(edge_index_flat.at[pl.ds(off, K)], sb, sem)
            pltpu.async_copy(edge_index_flat.at[pl.ds(E + off, K)], db, sem)
            pltpu.async_copy(gsc.at[pl.ds(off, K)], gb, sem)

        def wait_loads(sb, db, gb, sem):
            pltpu.make_async_copy(edge_index_flat.at[pl.ds(0, K)], sb, sem).wait()
            pltpu.make_async_copy(edge_index_flat.at[pl.ds(0, K)], db, sem).wait()
            pltpu.make_async_copy(gsc.at[pl.ds(0, K)], gb, sem).wait()

        def wait_store(fb, sem):
            pltpu.make_async_copy(fb, flows.at[pl.ds(0, K)], sem).wait()

        def flow_run(blk, r):
            return pl.ds(tbase + (blk * NR + r) * (B * 128), 128)

        def compute1(blk, sb, db, gb, fb, sem):
            @plsc.parallel_loop(0, K, step=LANES, unroll=4)
            def _(i):
                sl = pl.ds(i, LANES)
                hi = plsc.load_gather(big_buf, [sb[sl]])
                hj = plsc.load_gather(big_buf, [db[sl]])
                fb[sl] = gb[sl] * (hi - hj)

            for r in range(NR):
                pltpu.async_copy(
                    fb.at[pl.ds(r * 128, 128)], flows.at[flow_run(blk, r)], sem)

        start_loads1(0, s0, d0, g0, lsem0)
        pltpu.sync_copy(heads_flat.at[pl.ds(b * N, N)], big_buf)

        def outer1(o, u):
            for p, (sb, db, gb, fb, lsem, ssem) in enumerate(bufs):
                blk = 2 * o + p
                nsb, ndb, ngb, _, nlsem, _ = bufs[1 - p]

                @pl.when(blk + 1 < NB)
                def _():
                    start_loads1(blk + 1, nsb, ndb, ngb, nlsem)

                wait_loads(sb, db, gb, lsem)

                @pl.when(o > 0)
                def _():
                    wait_store(fb, ssem)

                compute1(blk, sb, db, gb, fb, ssem)
            return u

        lax.fori_loop(0, NO, outer1, 0)
        wait_store(f0, ssem0)
        wait_store(f1, ssem1)

        def start_loads2(blk, sb, db, fb, sem):
            off = blk * K
            pltpu.async_copy(edge_index_flat.at[pl.ds(qbase + off, K)], sb, sem)
            pltpu.async_copy(
                edge_index_flat.at[pl.ds(E + qbase + off, K)], db, sem)
            for r in range(NR):
                pltpu.async_copy(
                    flows.at[flow_run(blk, r)], fb.at[pl.ds(r * 128, 128)], sem)

        def wait_loads2(sb, db, fb, sem):
            pltpu.make_async_copy(edge_index_flat.at[pl.ds(0, K)], sb, sem).wait()
            pltpu.make_async_copy(edge_index_flat.at[pl.ds(0, K)], db, sem).wait()
            pltpu.make_async_copy(flows.at[pl.ds(0, K)], fb, sem).wait()

        start_loads2(0, s0, d0, f0, lsem0)

        zeros16 = jnp.zeros((LANES,), jnp.float32)

        @plsc.parallel_loop(0, N, step=LANES, unroll=8)
        def _(i):
            big_buf[pl.ds(i, LANES)] = zeros16

        def outer2(o, u):
            for p, (sb, db, gb, fb, lsem, ssem) in enumerate(bufs):
                blk = 2 * o + p
                nsb, ndb, _, nfb, nlsem, _ = bufs[1 - p]

                @pl.when(blk + 1 < NB)
                def _():
                    start_loads2(blk + 1, nsb, ndb, nfb, nlsem)

                wait_loads2(sb, db, fb, lsem)

                @plsc.parallel_loop(0, K, step=LANES, unroll=4)
                def _(i):
                    sl = pl.ds(i, LANES)
                    f16 = fb[sl]
                    plsc.addupdate_scatter(big_buf, [sb[sl]], f16)
                    plsc.addupdate_scatter(big_buf, [db[sl]], -f16)
            return u

        lax.fori_loop(0, NO, outer2, 0)
        pltpu.sync_copy(big_buf, partials.at[pl.ds((q * B + b) * N, N)])

    return main


def _build_reduce(B, N, KB):
    NQ = (B * N) // (NC * NS)
    assert N % 4 == 0 and NQ == N // 4 and NQ % KB == 0
    KBP = ((KB + LANES - 1) // LANES) * LANES

    mesh = plsc.VectorSubcoreMesh(
        core_axis_name="c", subcore_axis_name="s",
        num_cores=NC, num_subcores=NS)

    @functools.partial(
        pl.kernel,
        out_type=[
            jax.ShapeDtypeStruct((NC * NS * LANES,), jnp.float32),
            jax.ShapeDtypeStruct((LANES,), jnp.float32),
        ],
        mesh=mesh,
        scratch_types=[
            pltpu.VMEM((4 * KBP,), jnp.float32),
            pltpu.VMEM((KBP,), jnp.float32),
            pltpu.VMEM((LANES,), jnp.float32),
            pltpu.VMEM((B * LANES,), jnp.float32),
        ],
        interpret=False,
    )
    def reduce(partials, demands, heads_flat, cont_out, bound_out,
               p_blk, d_blk, acc_buf, bbuf):
        c = lax.axis_index("c")
        s = lax.axis_index("s")
        w = c * NS + s
        b = w // 4
        nbase = b * N + (w % 4) * NQ
        zeros16 = jnp.zeros((LANES,), jnp.float32)

        if KBP != KB:
            for qi in range(4):
                p_blk[pl.ds(qi * KBP + KBP - LANES, LANES)] = zeros16
            d_blk[pl.ds(KBP - LANES, LANES)] = zeros16

        def block(blk, acc):
            off = nbase + blk * KB
            for qi in range(4):
                pltpu.sync_copy(partials.at[pl.ds(qi * B * N + off, KB)],
                                p_blk.at[pl.ds(qi * KBP, KB)])
            pltpu.sync_copy(demands.at[pl.ds(off, KB)], d_blk.at[pl.ds(0, KB)])

            def vec(j, a):
                base = j * LANES
                v = ((p_blk[pl.ds(base, LANES)]
                      + p_blk[pl.ds(KBP + base, LANES)])
                     + (p_blk[pl.ds(2 * KBP + base, LANES)]
                        + p_blk[pl.ds(3 * KBP + base, LANES)])) \
                    - d_blk[pl.ds(base, LANES)]
                return a + v * v

            return lax.fori_loop(0, KBP // LANES, vec, acc)

        acc = lax.fori_loop(0, NQ // KB, block, zeros16)
        acc_buf[...] = acc
        pltpu.sync_copy(acc_buf, cont_out.at[pl.ds(w * LANES, LANES)])

        @pl.when(w == 0)
        def _():
            for bi in range(B):
                pltpu.sync_copy(heads_flat.at[pl.ds(bi * N, LANES)],
                                bbuf.at[pl.ds(bi * LANES, LANES)])
            lane = lax.iota(jnp.int32, LANES)
            m4 = lane < 4
            bacc = zeros16
            for bi in range(B):
                r = bbuf[pl.ds(bi * LANES, LANES)] - RESERVOIR_HEAD
                bacc = bacc + jnp.where(m4, r * r, 0.0)
            acc_buf[...] = bacc
            pltpu.sync_copy(acc_buf, bound_out)

    return reduce


def kernel(node_heads, demands, edge_index, edge_attr):
    B, N = node_heads.shape
    E = edge_index.shape[1]
    main = _build_main(B, N, E, K=3200)
    reduce = _build_reduce(B, N, KB=1000)
    heads_flat = node_heads.reshape(B * N)
    flows_flat, partials = main(
        heads_flat, edge_index.reshape(2 * E), edge_attr[:, 0])
    cont_p, bound_p = reduce(partials, demands.reshape(B * N), heads_flat)
    continuity = jnp.sum(cont_p) / (B * N)
    boundary = jnp.sum(bound_p) / (B * 4)
    total = continuity + boundary
    flows = flows_flat.reshape(E // 128, B, 128).transpose(1, 0, 2).reshape(B, E)
    return (continuity, boundary, total, flows)


# speedup vs baseline: 64.5245x; 1.0004x over previous
import functools

import jax
import jax.numpy as jnp
from jax import lax
from jax.experimental import pallas as pl
from jax.experimental.pallas import tpu as pltpu
from jax.experimental.pallas import tpu_sc as plsc

NC = 2
NS = 16
LANES = 16
RESERVOIR_HEAD = 50.0


def _vec_loop(n_vec, body):
    lax.fori_loop(0, n_vec, lambda j, c: (body(j), c)[1], 0)


def _build_main(B, N, E, K):
    EQ = E // 4
    assert EQ % K == 0 and K % LANES == 0 and N % LANES == 0

    mesh = plsc.VectorSubcoreMesh(
        core_axis_name="c", subcore_axis_name="s",
        num_cores=NC, num_subcores=NS)

    NB = EQ // K
    assert NB % 2 == 0 and K % 128 == 0
    NO = NB // 2
    NR = K // 128

    @functools.partial(
        pl.kernel,
        out_type=[
            jax.ShapeDtypeStruct((B * E,), jnp.float32),
            jax.ShapeDtypeStruct((4 * B * N,), jnp.float32),
        ],
        mesh=mesh,
        scratch_types=[
            pltpu.VMEM((N,), jnp.float32),
            pltpu.VMEM((K,), jnp.float32),
            pltpu.VMEM((K,), jnp.float32),
            pltpu.VMEM((K,), jnp.int32),
            pltpu.VMEM((K,), jnp.int32),
            pltpu.VMEM((K,), jnp.int32),
            pltpu.VMEM((K,), jnp.int32),
            pltpu.VMEM((K,), jnp.float32),
            pltpu.VMEM((K,), jnp.float32),
            pltpu.SemaphoreType.DMA,
            pltpu.SemaphoreType.DMA,
            pltpu.SemaphoreType.DMA,
            pltpu.SemaphoreType.DMA,
        ],
        compiler_params=pltpu.CompilerParams(needs_layout_passes=False),
        interpret=False,
    )
    def main(heads_flat, edge_index_flat, gsc, flows, partials,
             big_buf, g0, g1, s0, s1, d0, d1, f0, f1,
             lsem0, lsem1, ssem0, ssem1):
        c = lax.axis_index("c")
        s = lax.axis_index("s")
        b = s % B
        q = 2 * c + s // B
        qbase = q * EQ
        tbase = (qbase // 128) * (B * 128) + b * 128
        bufs = ((s0, d0, g0, f0, lsem0, ssem0),
                (s1, d1, g1, f1, lsem1, ssem1))

        def start_loads1(blk, sb, db, gb, sem):
            off = qbase + blk * K
            pltpu.async_copy(edge_index_flat.at[pl.ds(off, K)], sb, sem)
            pltpu.async_copy(edge_index_flat.at[pl.ds(E + off, K)], db, sem)
            pltpu.async_copy(gsc.at[pl.ds(off, K)], gb, sem)

        def wait_loads(sb, db, gb, sem):
            pltpu.make_async_copy(edge_index_flat.at[pl.ds(0, K)], sb, sem).wait()
            pltpu.make_async_copy(edge_index_flat.at[pl.ds(0, K)], db, sem).wait()
            pltpu.make_async_copy(gsc.at[pl.ds(0, K)], gb, sem).wait()

        def wait_store(fb, sem):
            pltpu.make_async_copy(fb, flows.at[pl.ds(0, K)], sem).wait()

        def flow_run(blk, r):
            return pl.ds(tbase + (blk * NR + r) * (B * 128), 128)

        def compute1(blk, sb, db, gb, fb, sem):
            @plsc.parallel_loop(0, K, step=LANES, unroll=8)
            def _(i):
                sl = pl.ds(i, LANES)
                hi = plsc.load_gather(big_buf, [sb[sl]])
                hj = plsc.load_gather(big_buf, [db[sl]])
                fb[sl] = gb[sl] * (hi - hj)

            for r in range(NR):
                pltpu.async_copy(
                    fb.at[pl.ds(r * 128, 128)], flows.at[flow_run(blk, r)], sem)

        start_loads1(0, s0, d0, g0, lsem0)
        pltpu.sync_copy(heads_flat.at[pl.ds(b * N, N)], big_buf)

        def outer1(o, u):
            for p, (sb, db, gb, fb, lsem, ssem) in enumerate(bufs):
                blk = 2 * o + p
                nsb, ndb, ngb, _, nlsem, _ = bufs[1 - p]

                @pl.when(blk + 1 < NB)
                def _():
                    start_loads1(blk + 1, nsb, ndb, ngb, nlsem)

                wait_loads(sb, db, gb, lsem)

                @pl.when(o > 0)
                def _():
                    wait_store(fb, ssem)

                compute1(blk, sb, db, gb, fb, ssem)
            return u

        lax.fori_loop(0, NO, outer1, 0)
        wait_store(f0, ssem0)
        wait_store(f1, ssem1)

        def start_loads2(blk, sb, db, fb, sem):
            off = blk * K
            pltpu.async_copy(edge_index_flat.at[pl.ds(qbase + off, K)], sb, sem)
            pltpu.async_copy(
                edge_index_flat.at[pl.ds(E + qbase + off, K)], db, sem)
            for r in range(NR):
                pltpu.async_copy(
                    flows.at[flow_run(blk, r)], fb.at[pl.ds(r * 128, 128)], sem)

        def wait_loads2(sb, db, fb, sem):
            pltpu.make_async_copy(edge_index_flat.at[pl.ds(0, K)], sb, sem).wait()
            pltpu.make_async_copy(edge_index_flat.at[pl.ds(0, K)], db, sem).wait()
            pltpu.make_async_copy(flows.at[pl.ds(0, K)], fb, sem).wait()

        start_loads2(0, s0, d0, f0, lsem0)

        zeros16 = jnp.zeros((LANES,), jnp.float32)

        @plsc.parallel_loop(0, N, step=LANES, unroll=8)
        def _(i):
            big_buf[pl.ds(i, LANES)] = zeros16

        def outer2(o, u):
            for p, (sb, db, gb, fb, lsem, ssem) in enumerate(bufs):
                blk = 2 * o + p
                nsb, ndb, _, nfb, nlsem, _ = bufs[1 - p]

                @pl.when(blk + 1 < NB)
                def _():
                    start_loads2(blk + 1, nsb, ndb, nfb, nlsem)

                wait_loads2(sb, db, fb, lsem)

                @plsc.parallel_loop(0, K, step=LANES, unroll=8)
                def _(i):
                    sl = pl.ds(i, LANES)
                    f16 = fb[sl]
                    plsc.addupdate_scatter(big_buf, [sb[sl]], f16)
                    plsc.addupdate_scatter(big_buf, [db[sl]], -f16)
            return u

        lax.fori_loop(0, NO, outer2, 0)
        pltpu.sync_copy(big_buf, partials.at[pl.ds((q * B + b) * N, N)])

    return main


def _build_reduce(B, N, KB):
    NQ = (B * N) // (NC * NS)
    assert N % 4 == 0 and NQ == N // 4 and NQ % KB == 0
    KBP = ((KB + LANES - 1) // LANES) * LANES

    mesh = plsc.VectorSubcoreMesh(
        core_axis_name="c", subcore_axis_name="s",
        num_cores=NC, num_subcores=NS)

    @functools.partial(
        pl.kernel,
        out_type=[
            jax.ShapeDtypeStruct((NC * NS * LANES,), jnp.float32),
            jax.ShapeDtypeStruct((LANES,), jnp.float32),
        ],
        mesh=mesh,
        scratch_types=[
            pltpu.VMEM((4 * KBP,), jnp.float32),
            pltpu.VMEM((KBP,), jnp.float32),
            pltpu.VMEM((LANES,), jnp.float32),
            pltpu.VMEM((B * LANES,), jnp.float32),
        ],
        interpret=False,
    )
    def reduce(partials, demands, heads_flat, cont_out, bound_out,
               p_blk, d_blk, acc_buf, bbuf):
        c = lax.axis_index("c")
        s = lax.axis_index("s")
        w = c * NS + s
        b = w // 4
        nbase = b * N + (w % 4) * NQ
        zeros16 = jnp.zeros((LANES,), jnp.float32)

        if KBP != KB:
            for qi in range(4):
                p_blk[pl.ds(qi * KBP + KBP - LANES, LANES)] = zeros16
            d_blk[pl.ds(KBP - LANES, LANES)] = zeros16

        def block(blk, acc):
            off = nbase + blk * KB
            for qi in range(4):
                pltpu.sync_copy(partials.at[pl.ds(qi * B * N + off, KB)],
                                p_blk.at[pl.ds(qi * KBP, KB)])
            pltpu.sync_copy(demands.at[pl.ds(off, KB)], d_blk.at[pl.ds(0, KB)])

            def vec(j, a):
                base = j * LANES
                v = ((p_blk[pl.ds(base, LANES)]
                      + p_blk[pl.ds(KBP + base, LANES)])
                     + (p_blk[pl.ds(2 * KBP + base, LANES)]
                        + p_blk[pl.ds(3 * KBP + base, LANES)])) \
                    - d_blk[pl.ds(base, LANES)]
                return a + v * v

            return lax.fori_loop(0, KBP // LANES, vec, acc)

        acc = lax.fori_loop(0, NQ // KB, block, zeros16)
        acc_buf[...] = acc
        pltpu.sync_copy(acc_buf, cont_out.at[pl.ds(w * LANES, LANES)])

        @pl.when(w == 0)
        def _():
            for bi in range(B):
                pltpu.sync_copy(heads_flat.at[pl.ds(bi * N, LANES)],
                                bbuf.at[pl.ds(bi * LANES, LANES)])
            lane = lax.iota(jnp.int32, LANES)
            m4 = lane < 4
            bacc = zeros16
            for bi in range(B):
                r = bbuf[pl.ds(bi * LANES, LANES)] - RESERVOIR_HEAD
                bacc = bacc + jnp.where(m4, r * r, 0.0)
            acc_buf[...] = bacc
            pltpu.sync_copy(acc_buf, bound_out)

    return reduce


def kernel(node_heads, demands, edge_index, edge_attr):
    B, N = node_heads.shape
    E = edge_index.shape[1]
    main = _build_main(B, N, E, K=3200)
    reduce = _build_reduce(B, N, KB=1000)
    heads_flat = node_heads.reshape(B * N)
    flows_flat, partials = main(
        heads_flat, edge_index.reshape(2 * E), edge_attr[:, 0])
    cont_p, bound_p = reduce(partials, demands.reshape(B * N), heads_flat)
    continuity = jnp.sum(cont_p) / (B * N)
    boundary = jnp.sum(bound_p) / (B * 4)
    total = continuity + boundary
    flows = flows_flat.reshape(E // 128, B, 128).transpose(1, 0, 2).reshape(B, E)
    return (continuity, boundary, total, flows)


# speedup vs baseline: 71.4597x; 1.1075x over previous
import functools

import jax
import jax.numpy as jnp
from jax import lax
from jax.experimental import pallas as pl
from jax.experimental.pallas import tpu as pltpu
from jax.experimental.pallas import tpu_sc as plsc

NC = 2
NS = 16
LANES = 16
RESERVOIR_HEAD = 50.0


def _vec_loop(n_vec, body):
    lax.fori_loop(0, n_vec, lambda j, c: (body(j), c)[1], 0)


def _build_main(B, N, E, K):
    EQ = E // 4
    assert EQ % K == 0 and K % LANES == 0 and N % LANES == 0

    mesh = plsc.VectorSubcoreMesh(
        core_axis_name="c", subcore_axis_name="s",
        num_cores=NC, num_subcores=NS)

    NB = EQ // K
    assert NB % 2 == 0 and K % 128 == 0
    NO = NB // 2
    NR = K // 128
    BPC = B // NC
    assert NS == 4 * BPC
    NQ4 = N // 4
    CH = 1000
    CHP = 1008
    NCH = NQ4 // CH
    assert NQ4 % CH == 0 and N % 4 == 0

    @functools.partial(
        pl.kernel,
        out_type=[
            jax.ShapeDtypeStruct((B * E,), jnp.float32),
            jax.ShapeDtypeStruct((NC * NS * LANES,), jnp.float32),
            jax.ShapeDtypeStruct((LANES,), jnp.float32),
            jax.ShapeDtypeStruct((NC * NS * N,), jnp.float32),
        ],
        mesh=mesh,
        scratch_types=[
            pltpu.VMEM((N,), jnp.float32),
            pltpu.VMEM((K,), jnp.float32),
            pltpu.VMEM((K,), jnp.float32),
            pltpu.VMEM((K,), jnp.int32),
            pltpu.VMEM((K,), jnp.int32),
            pltpu.VMEM((K,), jnp.int32),
            pltpu.VMEM((K,), jnp.int32),
            pltpu.VMEM((K,), jnp.float32),
            pltpu.VMEM((K,), jnp.float32),
            pltpu.SemaphoreType.DMA,
            pltpu.SemaphoreType.DMA,
            pltpu.SemaphoreType.DMA,
            pltpu.SemaphoreType.DMA,
        ],
        compiler_params=pltpu.CompilerParams(needs_layout_passes=False),
        interpret=False,
    )
    def main(heads_flat, edge_index_flat, gsc, demands_flat,
             flows, cont_out, bound_out, partials,
             big_buf, g0, g1, s0, s1, d0, d1, f0, f1,
             lsem0, lsem1, ssem0, ssem1):
        c = lax.axis_index("c")
        s = lax.axis_index("s")
        lb = s // 4
        b = c * BPC + lb
        q = s % 4
        qbase = q * EQ
        tbase = (qbase // 128) * (B * 128) + b * 128
        bufs = ((s0, d0, g0, f0, lsem0, ssem0),
                (s1, d1, g1, f1, lsem1, ssem1))

        def start_loads1(blk, sb, db, gb, sem):
            off = qbase + blk * K
            pltpu.async_copy(edge_index_flat.at[pl.ds(off, K)], sb, sem)
            pltpu.async_copy(edge_index_flat.at[pl.ds(E + off, K)], db, sem)
            pltpu.async_copy(gsc.at[pl.ds(off, K)], gb, sem)

        def wait_loads(sb, db, gb, sem):
            pltpu.make_async_copy(edge_index_flat.at[pl.ds(0, K)], sb, sem).wait()
            pltpu.make_async_copy(edge_index_flat.at[pl.ds(0, K)], db, sem).wait()
            pltpu.make_async_copy(gsc.at[pl.ds(0, K)], gb, sem).wait()

        def wait_store(fb, sem):
            pltpu.make_async_copy(fb, flows.at[pl.ds(0, K)], sem).wait()

        def flow_run(blk, r):
            return pl.ds(tbase + (blk * NR + r) * (B * 128), 128)

        def compute1(blk, sb, db, gb, fb, sem):
            @plsc.parallel_loop(0, K, step=LANES, unroll=8)
            def _(i):
                sl = pl.ds(i, LANES)
                hi = plsc.load_gather(big_buf, [sb[sl]])
                hj = plsc.load_gather(big_buf, [db[sl]])
                fb[sl] = gb[sl] * (hi - hj)

            for r in range(NR):
                pltpu.async_copy(
                    fb.at[pl.ds(r * 128, 128)], flows.at[flow_run(blk, r)], sem)

        start_loads1(0, s0, d0, g0, lsem0)
        pltpu.sync_copy(heads_flat.at[pl.ds(b * N, N)], big_buf)

        def outer1(o, u):
            for p, (sb, db, gb, fb, lsem, ssem) in enumerate(bufs):
                blk = 2 * o + p
                nsb, ndb, ngb, _, nlsem, _ = bufs[1 - p]

                @pl.when(blk + 1 < NB)
                def _():
                    start_loads1(blk + 1, nsb, ndb, ngb, nlsem)

                wait_loads(sb, db, gb, lsem)

                @pl.when(o > 0)
                def _():
                    wait_store(fb, ssem)

                compute1(blk, sb, db, gb, fb, ssem)
            return u

        lax.fori_loop(0, NO, outer1, 0)
        wait_store(f0, ssem0)
        wait_store(f1, ssem1)

        def start_loads2(blk, sb, db, fb, sem):
            off = blk * K
            pltpu.async_copy(edge_index_flat.at[pl.ds(qbase + off, K)], sb, sem)
            pltpu.async_copy(
                edge_index_flat.at[pl.ds(E + qbase + off, K)], db, sem)
            for r in range(NR):
                pltpu.async_copy(
                    flows.at[flow_run(blk, r)], fb.at[pl.ds(r * 128, 128)], sem)

        def wait_loads2(sb, db, fb, sem):
            pltpu.make_async_copy(edge_index_flat.at[pl.ds(0, K)], sb, sem).wait()
            pltpu.make_async_copy(edge_index_flat.at[pl.ds(0, K)], db, sem).wait()
            pltpu.make_async_copy(flows.at[pl.ds(0, K)], fb, sem).wait()

        start_loads2(0, s0, d0, f0, lsem0)

        zeros16 = jnp.zeros((LANES,), jnp.float32)

        @plsc.parallel_loop(0, N, step=LANES, unroll=8)
        def _(i):
            big_buf[pl.ds(i, LANES)] = zeros16

        def outer2(o, u):
            for p, (sb, db, gb, fb, lsem, ssem) in enumerate(bufs):
                blk = 2 * o + p
                nsb, ndb, _, nfb, nlsem, _ = bufs[1 - p]

                @pl.when(blk + 1 < NB)
                def _():
                    start_loads2(blk + 1, nsb, ndb, nfb, nlsem)

                wait_loads2(sb, db, fb, lsem)

                @plsc.parallel_loop(0, K, step=LANES, unroll=8)
                def _(i):
                    sl = pl.ds(i, LANES)
                    f16 = fb[sl]
                    plsc.addupdate_scatter(big_buf, [sb[sl]], f16)
                    plsc.addupdate_scatter(big_buf, [db[sl]], -f16)
            return u

        lax.fori_loop(0, NO, outer2, 0)

        pltpu.sync_copy(big_buf, partials.at[pl.ds(((c * NS + lb * 4 + q) * N), N)])
        plsc.subcore_barrier()

        nb3 = (s % 4) * NQ4
        ACC = 10 * CHP

        def buf_off(t, r):
            return (2 * t + r) * CHP

        for t in range(5):
            for r in range(2):
                big_buf[pl.ds(buf_off(t, r) + CHP - LANES, LANES)] = zeros16
        big_buf[pl.ds(ACC, LANES)] = zeros16

        def start_loads3(ck, r, sem):
            off = nb3 + ck * CH
            for qi in range(4):
                pltpu.async_copy(
                    partials.at[pl.ds((c * NS + lb * 4 + qi) * N + off, CH)],
                    big_buf.at[pl.ds(buf_off(qi, r), CH)], sem)
            pltpu.async_copy(
                demands_flat.at[pl.ds(b * N + off, CH)],
                big_buf.at[pl.ds(buf_off(4, r), CH)], sem)

        def wait_loads3(r, sem):
            for t in range(5):
                pltpu.make_async_copy(
                    demands_flat.at[pl.ds(0, CH)],
                    big_buf.at[pl.ds(buf_off(t, r), CH)], sem).wait()

        def compute3(r):
            def vec(j, a):
                i = j * LANES
                v = ((big_buf[pl.ds(buf_off(0, r) + i, LANES)]
                      + big_buf[pl.ds(buf_off(1, r) + i, LANES)])
                     + (big_buf[pl.ds(buf_off(2, r) + i, LANES)]
                        + big_buf[pl.ds(buf_off(3, r) + i, LANES)])) \
                    - big_buf[pl.ds(buf_off(4, r) + i, LANES)]
                return a + v * v

            acc = lax.fori_loop(0, CHP // LANES, vec,
                                big_buf[pl.ds(ACC, LANES)])
            big_buf[pl.ds(ACC, LANES)] = acc

        start_loads3(0, 0, lsem0)

        def outer3(o, u):
            for p in (0, 1):
                ck = 2 * o + p
                sem = (lsem0, lsem1)[p]
                nsem = (lsem0, lsem1)[1 - p]

                @pl.when(ck < NCH)
                def _():
                    @pl.when(ck + 1 < NCH)
                    def _():
                        start_loads3(ck + 1, 1 - p, nsem)

                    wait_loads3(p, sem)
                    compute3(p)
            return u

        lax.fori_loop(0, (NCH + 1) // 2, outer3, 0)
        pltpu.sync_copy(big_buf.at[pl.ds(ACC, LANES)],
                        cont_out.at[pl.ds((c * NS + s) * LANES, LANES)])

        @pl.when(c * NS + s == 0)
        def _():
            for bi in range(B):
                pltpu.sync_copy(heads_flat.at[pl.ds(bi * N, LANES)],
                                big_buf.at[pl.ds(bi * LANES, LANES)])
            lane = lax.iota(jnp.int32, LANES)
            m4 = lane < 4
            bacc = zeros16
            for bi in range(B):
                rr = big_buf[pl.ds(bi * LANES, LANES)] - RESERVOIR_HEAD
                bacc = bacc + jnp.where(m4, rr * rr, 0.0)
            big_buf[pl.ds(ACC, LANES)] = bacc
            pltpu.sync_copy(big_buf.at[pl.ds(ACC, LANES)], bound_out)

    return main


def _unused_build_reduce(B, N, KB):
    NQ = (B * N) // (NC * NS)
    assert N % 4 == 0 and NQ == N // 4 and NQ % KB == 0
    KBP = ((KB + LANES - 1) // LANES) * LANES

    mesh = plsc.VectorSubcoreMesh(
        core_axis_name="c", subcore_axis_name="s",
        num_cores=NC, num_subcores=NS)

    @functools.partial(
        pl.kernel,
        out_type=[
            jax.ShapeDtypeStruct((NC * NS * LANES,), jnp.float32),
            jax.ShapeDtypeStruct((LANES,), jnp.float32),
        ],
        mesh=mesh,
        scratch_types=[
            pltpu.VMEM((4 * KBP,), jnp.float32),
            pltpu.VMEM((KBP,), jnp.float32),
            pltpu.VMEM((LANES,), jnp.float32),
            pltpu.VMEM((B * LANES,), jnp.float32),
        ],
        interpret=False,
    )
    def reduce(partials, demands, heads_flat, cont_out, bound_out,
               p_blk, d_blk, acc_buf, bbuf):
        c = lax.axis_index("c")
        s = lax.axis_index("s")
        w = c * NS + s
        b = w // 4
        nbase = b * N + (w % 4) * NQ
        zeros16 = jnp.zeros((LANES,), jnp.float32)

        if KBP != KB:
            for qi in range(4):
                p_blk[pl.ds(qi * KBP + KBP - LANES, LANES)] = zeros16
            d_blk[pl.ds(KBP - LANES, LANES)] = zeros16

        def block(blk, acc):
            off = nbase + blk * KB
            for qi in range(4):
                pltpu.sync_copy(partials.at[pl.ds(qi * B * N + off, KB)],
                                p_blk.at[pl.ds(qi * KBP, KB)])
            pltpu.sync_copy(demands.at[pl.ds(off, KB)], d_blk.at[pl.ds(0, KB)])

            def vec(j, a):
                base = j * LANES
                v = ((p_blk[pl.ds(base, LANES)]
                      + p_blk[pl.ds(KBP + base, LANES)])
                     + (p_blk[pl.ds(2 * KBP + base, LANES)]
                        + p_blk[pl.ds(3 * KBP + base, LANES)])) \
                    - d_blk[pl.ds(base, LANES)]
                return a + v * v

            return lax.fori_loop(0, KBP // LANES, vec, acc)

        acc = lax.fori_loop(0, NQ // KB, block, zeros16)
        acc_buf[...] = acc
        pltpu.sync_copy(acc_buf, cont_out.at[pl.ds(w * LANES, LANES)])

        @pl.when(w == 0)
        def _():
            for bi in range(B):
                pltpu.sync_copy(heads_flat.at[pl.ds(bi * N, LANES)],
                                bbuf.at[pl.ds(bi * LANES, LANES)])
            lane = lax.iota(jnp.int32, LANES)
            m4 = lane < 4
            bacc = zeros16
            for bi in range(B):
                r = bbuf[pl.ds(bi * LANES, LANES)] - RESERVOIR_HEAD
                bacc = bacc + jnp.where(m4, r * r, 0.0)
            acc_buf[...] = bacc
            pltpu.sync_copy(acc_buf, bound_out)

    return reduce


def kernel(node_heads, demands, edge_index, edge_attr):
    B, N = node_heads.shape
    E = edge_index.shape[1]
    main = _build_main(B, N, E, K=3200)
    heads_flat = node_heads.reshape(B * N)
    flows_flat, cont_p, bound_p, _ = main(
        heads_flat, edge_index.reshape(2 * E), edge_attr[:, 0],
        demands.reshape(B * N))
    continuity = jnp.sum(cont_p) / (B * N)
    boundary = jnp.sum(bound_p) / (B * 4)
    total = continuity + boundary
    flows = flows_flat.reshape(E // 128, B, 128).transpose(1, 0, 2).reshape(B, E)
    return (continuity, boundary, total, flows)
